# baseline (speedup 1.0000x reference)
import functools

import jax
import jax.numpy as jnp
from jax import lax
from jax.experimental import pallas as pl
from jax.experimental.pallas import tpu as pltpu
from jax.experimental.pallas import tpu_sc as plsc

B = 4096
L = 50
DIM = 64
NC = 2
NS = 16
NW = NC * NS

BPW = B // NW
CB = 2
NCHUNK = BPW // CB
ROWS = CB * L
ROWS_PAD = 104
NVEC = DIM // 16


NBUF = 8


def _body(sent_hbm, table_hbm, out_hbm, idx_v, buf_v, out_v, *sems):
  wid = lax.axis_index("s") * NC + lax.axis_index("c")

  pltpu.sync_copy(sent_hbm.at[wid], idx_v)

  def start(c, b):
    pltpu.async_copy(table_hbm.at[idx_v.at[c]], buf_v.at[b], sems[b])

  def wait(c, b):
    pltpu.make_async_copy(table_hbm.at[idx_v.at[c]], buf_v.at[b], sems[b]).wait()

  for b in range(NBUF):
    start(b, b)

  def compute(c, b):
    for i in range(CB):
      accs = [buf_v[b, i * L, pl.ds(16 * j, 16)] for j in range(NVEC)]
      for l in range(1, L):
        for j in range(NVEC):
          accs[j] = accs[j] + buf_v[b, i * L + l, pl.ds(16 * j, 16)]
      row = CB * c + i
      for j in range(NVEC):
        out_v[row, pl.ds(16 * j, 16)] = accs[j]

  def g_body(g, carry):
    for b in range(NBUF):
      c = NBUF * g + b
      wait(c, b)
      compute(c, b)

      @pl.when(g < NCHUNK // NBUF - 1)
      def _():
        start(c + NBUF, b)
    return carry

  lax.fori_loop(0, NCHUNK // NBUF, g_body, 0)

  pltpu.sync_copy(out_v, out_hbm.at[pl.ds(wid * BPW, BPW)])


@jax.jit
def _encode(sent3, table):
  mesh = plsc.VectorSubcoreMesh(core_axis_name="c", subcore_axis_name="s")
  return pl.kernel(
      _body,
      out_type=jax.ShapeDtypeStruct((B, DIM), jnp.float32),
      mesh=mesh,
      scratch_types=[
          pltpu.VMEM((NCHUNK, ROWS_PAD), jnp.int32),
          pltpu.VMEM((NBUF, ROWS_PAD, DIM), jnp.float32),
          pltpu.VMEM((BPW, DIM), jnp.float32),
      ] + [pltpu.SemaphoreType.DMA] * NBUF,
      compiler_params=pltpu.CompilerParams(use_tc_tiling_on_sc=False),
  )(sent3, table)


def kernel(sent, table):
  sent3 = sent.reshape(NW, NCHUNK, ROWS)
  sent3 = jnp.pad(sent3, ((0, 0), (0, 0), (0, ROWS_PAD - ROWS)))
  return _encode(sent3, table)


# speedup vs baseline: 1.1890x; 1.1890x over previous
import functools

import jax
import jax.numpy as jnp
from jax import lax
from jax.experimental import pallas as pl
from jax.experimental.pallas import tpu as pltpu
from jax.experimental.pallas import tpu_sc as plsc

B = 4096
L = 50
DIM = 64
NC = 2
NS = 16
NW = NC * NS

BPW = B // NW
CB = 2
NCHUNK = BPW // CB
ROWS = CB * L
ROWS_PAD = 100
NVEC = DIM // 16


NBUF = 8


def _body(sent_hbm, table_hbm, out_hbm, idx_v, buf_v, out_v, *sems):
  wid = lax.axis_index("s") * NC + lax.axis_index("c")

  pltpu.sync_copy(sent_hbm.at[wid], idx_v)

  def start(c, b):
    pltpu.async_copy(table_hbm.at[idx_v.at[c]], buf_v.at[b], sems[b])

  def wait(c, b):
    pltpu.make_async_copy(table_hbm.at[idx_v.at[c]], buf_v.at[b], sems[b]).wait()

  for b in range(NBUF):
    start(b, b)

  def compute(c, b):
    for i in range(CB):
      accs = [buf_v[b, i * L, pl.ds(16 * j, 16)] for j in range(NVEC)]
      for l in range(1, L):
        for j in range(NVEC):
          accs[j] = accs[j] + buf_v[b, i * L + l, pl.ds(16 * j, 16)]
      row = CB * c + i
      for j in range(NVEC):
        out_v[row, pl.ds(16 * j, 16)] = accs[j]

  def g_body(g, carry):
    for b in range(NBUF):
      c = NBUF * g + b
      wait(c, b)
      compute(c, b)

      @pl.when(g < NCHUNK // NBUF - 1)
      def _():
        start(c + NBUF, b)
    return carry

  lax.fori_loop(0, NCHUNK // NBUF, g_body, 0)

  pltpu.sync_copy(out_v, out_hbm.at[pl.ds(wid * BPW, BPW)])


@jax.jit
def _encode(sent3, table):
  mesh = plsc.VectorSubcoreMesh(core_axis_name="c", subcore_axis_name="s")
  return pl.kernel(
      _body,
      out_type=jax.ShapeDtypeStruct((B, DIM), jnp.float32),
      mesh=mesh,
      scratch_types=[
          pltpu.VMEM((NCHUNK, ROWS_PAD), jnp.int32),
          pltpu.VMEM((NBUF, ROWS_PAD, DIM), jnp.float32),
          pltpu.VMEM((BPW, DIM), jnp.float32),
      ] + [pltpu.SemaphoreType.DMA] * NBUF,
      compiler_params=pltpu.CompilerParams(use_tc_tiling_on_sc=False),
  )(sent3, table)


def kernel(sent, table):
  sent3 = sent.reshape(NW, NCHUNK, ROWS)
  return _encode(sent3, table)


# speedup vs baseline: 1.2078x; 1.0157x over previous
import jax
import jax.numpy as jnp
from jax import lax
from jax.experimental import pallas as pl
from jax.experimental.pallas import tpu as pltpu
from jax.experimental.pallas import tpu_sc as plsc

B = 4096
L = 50
DIM = 64
NC = 2
NS = 16
NW = NC * NS

BPW = B // NW
NVEC = DIM // 16
NBUF = 8


def _body(sent_hbm, table_hbm, out_hbm, idx_v, buf_v, out_v, *sems):
  wid = lax.axis_index("s") * NC + lax.axis_index("c")

  pltpu.sync_copy(sent_hbm.at[pl.ds(wid * BPW, BPW)], idx_v)

  def start(c, b):
    pltpu.async_copy(table_hbm.at[idx_v.at[c]], buf_v.at[b], sems[b])

  def wait(c, b):
    pltpu.make_async_copy(table_hbm.at[idx_v.at[c]], buf_v.at[b], sems[b]).wait()

  for b in range(NBUF):
    start(b, b)

  def compute(c, b):
    accs = [buf_v[b, 0, pl.ds(16 * j, 16)] for j in range(NVEC)]
    for l in range(1, L):
      for j in range(NVEC):
        accs[j] = accs[j] + buf_v[b, l, pl.ds(16 * j, 16)]
    for j in range(NVEC):
      out_v[c, pl.ds(16 * j, 16)] = accs[j]

  def g_body(g, carry):
    for b in range(NBUF):
      c = NBUF * g + b
      wait(c, b)
      compute(c, b)

      @pl.when(g < BPW // NBUF - 1)
      def _():
        start(c + NBUF, b)
    return carry

  lax.fori_loop(0, BPW // NBUF, g_body, 0)

  pltpu.sync_copy(out_v, out_hbm.at[pl.ds(wid * BPW, BPW)])


@jax.jit
def _encode(sent, table):
  mesh = plsc.VectorSubcoreMesh(core_axis_name="c", subcore_axis_name="s")
  return pl.kernel(
      _body,
      out_type=jax.ShapeDtypeStruct((B, DIM), jnp.float32),
      mesh=mesh,
      scratch_types=[
          pltpu.VMEM((BPW, L), jnp.int32),
          pltpu.VMEM((NBUF, L, DIM), jnp.float32),
          pltpu.VMEM((BPW, DIM), jnp.float32),
      ] + [pltpu.SemaphoreType.DMA] * NBUF,
      compiler_params=pltpu.CompilerParams(use_tc_tiling_on_sc=False),
  )(sent, table)


def kernel(sent, table):
  return _encode(sent, table)
